# baseline (speedup 1.0000x reference)
import jax
import jax.numpy as jnp
from jax import lax
from jax.experimental import pallas as pl
from jax.experimental.pallas import tpu as pltpu
from jax.experimental.pallas import tpu_sc as plsc

VDJ_DIM = 64
VOCAB = 1000
VOCAB_PAD = 1024
BATCH = 16384
NUM_TABLES = 5
TAB_WORDS = VDJ_DIM * VOCAB_PAD
T_WORDS = NUM_TABLES * TAB_WORDS
HALF_D = 32
HALF_WORDS = HALF_D * VOCAB_PAD
N_HALF = NUM_TABLES * 2
STREAM_D = 14

_NC = 2
_NS = 16
_NW = _NC * _NS
_BPW = BATCH // _NW
_BC = _BPW // 128
_SLAB = _BC * 8 * 128
_L = 16


def _gather_body(xp_hbm, tab_hbm, o0, o1, o2, o3, o4,
                 spm, idx_v, th0, th1, sl0, sl1, st0, st1, sw0, sw1, sg):
    outs = (o0, o1, o2, o3, o4)
    ths = (th0, th1)
    sts = (st0, st1)
    slabs = (sl0, sl1)
    sws = (sw0, sw1)
    cid = lax.axis_index("c")
    sid = lax.axis_index("s")
    wid = sid * _NC + cid

    fill = T_WORDS // _NS
    pltpu.sync_copy(tab_hbm.at[pl.ds(sid * fill, fill)],
                    spm.at[pl.ds(sid * fill, fill)])
    pltpu.sync_copy(xp_hbm.at[:, pl.ds(wid, 1)], idx_v)
    plsc.subcore_barrier()

    def load_half(h):
        return pltpu.async_copy(
            spm.at[pl.ds(h * HALF_WORDS, HALF_WORDS)],
            ths[h % 2], sts[h % 2])

    loads = [None] * N_HALF
    writes = [None, None]
    loads[0] = load_half(0)
    for h in range(N_HALF):
        t = h // 2
        if h + 1 < N_HALF:
            loads[h + 1] = load_half(h + 1)
        loads[h].wait()
        slab = slabs[h % 2]
        if writes[h % 2] is not None:
            for w in writes[h % 2]:
                w.wait()
        th = ths[h % 2]
        half_d0_g = (h % 2) * HALF_D

        engine = []
        for dl in range(STREAM_D):
            row = spm.at[pl.ds(t * TAB_WORDS + (half_d0_g + dl) * VOCAB_PAD,
                               VOCAB_PAD)]
            off = (dl >> 3) * _SLAB + (dl & 7) * 128
            for bc in range(_BC):
                engine.append(pltpu.async_copy(
                    row.at[idx_v.at[t, 0, 0, pl.ds(bc * 128, 128)]],
                    slab.at[pl.ds(bc * 1024 + off, 128)], sg))

        @plsc.parallel_loop(0, (_BPW // _L) * (HALF_D - STREAM_D), 1,
                            unroll=16)
        def _gather_loop(i):
            dl = (i >> 5) + STREAM_D
            j = i & 31
            xv = idx_v[t, 0, 0, pl.ds(j * _L, _L)]
            v = plsc.load_gather(th, [xv + dl * VOCAB_PAD])
            dyn = (j >> 3) * 1024 + (j & 7) * _L
            off = (dl >> 3) * _SLAB + (dl & 7) * 128
            slab[pl.ds(dyn + off, _L)] = v

        for g in engine:
            g.wait()

        half_d0 = (h % 2) * HALF_D
        ws = []
        for g in range(HALF_D // 8):
            dhi = half_d0 // 8 + g
            off = (dhi * 128 + wid * _BC) * 1024
            ws.append(pltpu.async_copy(
                slab.at[pl.ds(g * _SLAB, _SLAB)],
                outs[t].at[pl.ds(off, _SLAB)], sws[h % 2]))
        writes[h % 2] = ws
    for w in writes[0]:
        w.wait()
    for w in writes[1]:
        w.wait()


@jax.jit
def _vdj_gather(x, w0, w1, w2, w3, w4):
    xp = x.astype(jnp.int32).T.reshape(NUM_TABLES, _NW, 1, _BPW)
    pad = lambda w: jnp.pad(w.T, ((0, 0), (0, VOCAB_PAD - VOCAB))).reshape(-1)
    tab = jnp.concatenate([pad(w) for w in (w0, w1, w2, w3, w4)])

    kern = pl.kernel(
        _gather_body,
        out_type=tuple(
            jax.ShapeDtypeStruct((BATCH * VDJ_DIM,), jnp.float32)
            for _ in range(NUM_TABLES)
        ),
        mesh=plsc.VectorSubcoreMesh(core_axis_name="c", subcore_axis_name="s"),
        scratch_types=[
            pltpu.VMEM_SHARED((T_WORDS,), jnp.float32),
            pltpu.VMEM((NUM_TABLES, 1, 1, _BPW), jnp.int32),
            pltpu.VMEM((HALF_WORDS,), jnp.float32),
            pltpu.VMEM((HALF_WORDS,), jnp.float32),
            pltpu.VMEM((HALF_D // 8 * _SLAB,), jnp.float32),
            pltpu.VMEM((HALF_D // 8 * _SLAB,), jnp.float32),
            pltpu.SemaphoreType.DMA,
            pltpu.SemaphoreType.DMA,
            pltpu.SemaphoreType.DMA,
            pltpu.SemaphoreType.DMA,
            pltpu.SemaphoreType.DMA,
        ],
        compiler_params=pltpu.CompilerParams(needs_layout_passes=False),
    )
    outs = kern(xp, tab)
    return tuple(
        o.reshape(8, 128, 8, 128).transpose(1, 3, 0, 2).reshape(BATCH, VDJ_DIM)
        for o in outs
    )


def kernel(x, W_v_alpha, W_j_alpha, W_v_beta, W_d_beta, W_j_beta):
    return _vdj_gather(x, W_v_alpha, W_j_alpha, W_v_beta, W_d_beta, W_j_beta)


# speedup vs baseline: 1.2133x; 1.2133x over previous
import jax
import jax.numpy as jnp
from jax import lax
from jax.experimental import pallas as pl
from jax.experimental.pallas import tpu as pltpu
from jax.experimental.pallas import tpu_sc as plsc

VDJ_DIM = 64
VOCAB = 1000
VOCAB_PAD = 1024
BATCH = 16384
NUM_TABLES = 5
TAB_WORDS = VDJ_DIM * VOCAB_PAD
T_WORDS = NUM_TABLES * TAB_WORDS
HALF_D = 32
HALF_WORDS = HALF_D * VOCAB_PAD
N_HALF = NUM_TABLES * 2

_NC = 2
_NS = 16
_NW = _NC * _NS
_BPW = BATCH // _NW
_BC = _BPW // 128
_SLAB = _BC * 8 * 128
_L = 16


def _gather_body(xp_hbm, tab_hbm, o0, o1, o2, o3, o4,
                 spm, idx_v, th0, th1, sl0, sl1, st0, st1, sw0, sw1):
    outs = (o0, o1, o2, o3, o4)
    ths = (th0, th1)
    sts = (st0, st1)
    slabs = (sl0, sl1)
    sws = (sw0, sw1)
    cid = lax.axis_index("c")
    sid = lax.axis_index("s")
    wid = sid * _NC + cid

    fill = T_WORDS // _NS
    pltpu.sync_copy(tab_hbm.at[pl.ds(sid * fill, fill)],
                    spm.at[pl.ds(sid * fill, fill)])
    pltpu.sync_copy(xp_hbm.at[:, pl.ds(wid, 1)], idx_v)
    plsc.subcore_barrier()

    def load_half(h):
        return pltpu.async_copy(
            spm.at[pl.ds(h * HALF_WORDS, HALF_WORDS)],
            ths[h % 2], sts[h % 2])

    loads = [None] * N_HALF
    writes = [None, None]
    loads[0] = load_half(0)
    for h in range(N_HALF):
        t = h // 2
        if h + 1 < N_HALF:
            loads[h + 1] = load_half(h + 1)
        loads[h].wait()
        slab = slabs[h % 2]
        if writes[h % 2] is not None:
            for w in writes[h % 2]:
                w.wait()
        th = ths[h % 2]

        @plsc.parallel_loop(0, (_BPW // _L) * HALF_D, 1, unroll=16)
        def _gather_loop(i):
            j = i >> 5
            dl = i & (HALF_D - 1)
            xv = idx_v[t, 0, 0, pl.ds(j * _L, _L)]
            v = plsc.load_gather(th, [xv + dl * VOCAB_PAD])
            dyn = (j >> 3) * 1024 + (j & 7) * _L
            off = (dl >> 3) * _SLAB + (dl & 7) * 128
            slab[pl.ds(dyn + off, _L)] = v

        half_d0 = (h % 2) * HALF_D
        ws = []
        for g in range(HALF_D // 8):
            dhi = half_d0 // 8 + g
            off = (dhi * 128 + wid * _BC) * 1024
            ws.append(pltpu.async_copy(
                slab.at[pl.ds(g * _SLAB, _SLAB)],
                outs[t].at[pl.ds(off, _SLAB)], sws[h % 2]))
        writes[h % 2] = ws
    for w in writes[0]:
        w.wait()
    for w in writes[1]:
        w.wait()


@jax.jit
def _vdj_gather(x, w0, w1, w2, w3, w4):
    xp = x.astype(jnp.int32).T.reshape(NUM_TABLES, _NW, 1, _BPW)
    pad = lambda w: jnp.pad(w.T, ((0, 0), (0, VOCAB_PAD - VOCAB))).reshape(-1)
    tab = jnp.concatenate([pad(w) for w in (w0, w1, w2, w3, w4)])

    kern = pl.kernel(
        _gather_body,
        out_type=tuple(
            jax.ShapeDtypeStruct((BATCH * VDJ_DIM,), jnp.float32)
            for _ in range(NUM_TABLES)
        ),
        mesh=plsc.VectorSubcoreMesh(core_axis_name="c", subcore_axis_name="s"),
        scratch_types=[
            pltpu.VMEM_SHARED((T_WORDS,), jnp.float32),
            pltpu.VMEM((NUM_TABLES, 1, 1, _BPW), jnp.int32),
            pltpu.VMEM((HALF_WORDS,), jnp.float32),
            pltpu.VMEM((HALF_WORDS,), jnp.float32),
            pltpu.VMEM((HALF_D // 8 * _SLAB,), jnp.float32),
            pltpu.VMEM((HALF_D // 8 * _SLAB,), jnp.float32),
            pltpu.SemaphoreType.DMA,
            pltpu.SemaphoreType.DMA,
            pltpu.SemaphoreType.DMA,
            pltpu.SemaphoreType.DMA,
        ],
        compiler_params=pltpu.CompilerParams(needs_layout_passes=False),
    )
    outs = kern(xp, tab)
    return tuple(
        o.reshape(8, 128, 8, 128).transpose(1, 3, 0, 2).reshape(BATCH, VDJ_DIM)
        for o in outs
    )


def kernel(x, W_v_alpha, W_j_alpha, W_v_beta, W_d_beta, W_j_beta):
    return _vdj_gather(x, W_v_alpha, W_j_alpha, W_v_beta, W_d_beta, W_j_beta)


# speedup vs baseline: 1.2265x; 1.0109x over previous
import jax
import jax.numpy as jnp
from jax import lax
from jax.experimental import pallas as pl
from jax.experimental.pallas import tpu as pltpu
from jax.experimental.pallas import tpu_sc as plsc

VDJ_DIM = 64
VOCAB = 1000
VOCAB_PAD = 1024
BATCH = 16384
NUM_TABLES = 5
TAB_WORDS = VDJ_DIM * VOCAB_PAD
T_WORDS = NUM_TABLES * TAB_WORDS
HALF_D = 32
HALF_WORDS = HALF_D * VOCAB_PAD
N_HALF = NUM_TABLES * 2

_NC = 2
_NS = 16
_NW = _NC * _NS
_BPW = BATCH // _NW
_BC = _BPW // 128
_SLAB = _BC * 8 * 128
_L = 16


def _gather_body(xp_hbm, tab_hbm, o0, o1, o2, o3, o4,
                 spm, idx_v, th0, th1, sl0, sl1, st0, st1, sw0, sw1):
    outs = (o0, o1, o2, o3, o4)
    ths = (th0, th1)
    sts = (st0, st1)
    slabs = (sl0, sl1)
    sws = (sw0, sw1)
    cid = lax.axis_index("c")
    sid = lax.axis_index("s")
    wid = sid * _NC + cid

    fill = T_WORDS // _NS
    pltpu.sync_copy(tab_hbm.at[pl.ds(sid * fill, fill)],
                    spm.at[pl.ds(sid * fill, fill)])
    pltpu.sync_copy(xp_hbm.at[:, pl.ds(wid, 1)], idx_v)
    plsc.subcore_barrier()

    def load_half(h):
        return pltpu.async_copy(
            spm.at[pl.ds(h * HALF_WORDS, HALF_WORDS)],
            ths[h % 2], sts[h % 2])

    loads = [None] * N_HALF
    writes = [None, None]
    loads[0] = load_half(0)
    for h in range(N_HALF):
        t = h // 2
        if h + 1 < N_HALF:
            loads[h + 1] = load_half(h + 1)
        loads[h].wait()
        slab = slabs[h % 2]
        if writes[h % 2] is not None:
            for w in writes[h % 2]:
                w.wait()
        th = ths[h % 2]

        @plsc.parallel_loop(0, (_BPW // _L) * HALF_D, 1, unroll=8)
        def _gather_loop(i):
            j = i >> 5
            dl = i & (HALF_D - 1)
            xv = idx_v[t, 0, 0, pl.ds(j * _L, _L)]
            v = plsc.load_gather(th, [xv + dl * VOCAB_PAD])
            dyn = (j >> 3) * 1024 + (j & 7) * _L
            off = (dl >> 3) * _SLAB + (dl & 7) * 128
            slab[pl.ds(dyn + off, _L)] = v

        half_d0 = (h % 2) * HALF_D
        ws = []
        for g in range(HALF_D // 8):
            dhi = half_d0 // 8 + g
            off = (dhi * 128 + wid * _BC) * 1024
            ws.append(pltpu.async_copy(
                slab.at[pl.ds(g * _SLAB, _SLAB)],
                outs[t].at[pl.ds(off, _SLAB)], sws[h % 2]))
        writes[h % 2] = ws
    for w in writes[0]:
        w.wait()
    for w in writes[1]:
        w.wait()


@jax.jit
def _vdj_gather(x, w0, w1, w2, w3, w4):
    xp = x.astype(jnp.int32).T.reshape(NUM_TABLES, _NW, 1, _BPW)
    pad = lambda w: jnp.pad(w.T, ((0, 0), (0, VOCAB_PAD - VOCAB))).reshape(-1)
    tab = jnp.concatenate([pad(w) for w in (w0, w1, w2, w3, w4)])

    kern = pl.kernel(
        _gather_body,
        out_type=tuple(
            jax.ShapeDtypeStruct((BATCH * VDJ_DIM,), jnp.float32)
            for _ in range(NUM_TABLES)
        ),
        mesh=plsc.VectorSubcoreMesh(core_axis_name="c", subcore_axis_name="s"),
        scratch_types=[
            pltpu.VMEM_SHARED((T_WORDS,), jnp.float32),
            pltpu.VMEM((NUM_TABLES, 1, 1, _BPW), jnp.int32),
            pltpu.VMEM((HALF_WORDS,), jnp.float32),
            pltpu.VMEM((HALF_WORDS,), jnp.float32),
            pltpu.VMEM((HALF_D // 8 * _SLAB,), jnp.float32),
            pltpu.VMEM((HALF_D // 8 * _SLAB,), jnp.float32),
            pltpu.SemaphoreType.DMA,
            pltpu.SemaphoreType.DMA,
            pltpu.SemaphoreType.DMA,
            pltpu.SemaphoreType.DMA,
        ],
        compiler_params=pltpu.CompilerParams(needs_layout_passes=False),
    )
    outs = kern(xp, tab)
    return tuple(
        o.reshape(8, 128, 8, 128).transpose(1, 3, 0, 2).reshape(BATCH, VDJ_DIM)
        for o in outs
    )


def kernel(x, W_v_alpha, W_j_alpha, W_v_beta, W_d_beta, W_j_beta):
    return _vdj_gather(x, W_v_alpha, W_j_alpha, W_v_beta, W_d_beta, W_j_beta)
